# baseline (speedup 1.0000x reference)
import functools

import jax
import jax.numpy as jnp
from jax import lax
from jax.experimental import pallas as pl
from jax.experimental.pallas import tpu as pltpu
from jax.experimental.pallas import tpu_sc as plsc

BATCH = 16384
D = 64
L = 16
NW = 32
BPW = BATCH // NW
CH = 128
NCH = BPW // CH

_mesh = plsc.VectorSubcoreMesh(core_axis_name="c", subcore_axis_name="s")


@functools.partial(
    pl.kernel,
    mesh=_mesh,
    out_type=(
        jax.ShapeDtypeStruct((BATCH,), jnp.float32),
        jax.ShapeDtypeStruct((BATCH,), jnp.float32),
    ),
    scratch_types=[
        pltpu.VMEM((NCH, CH), jnp.int32),
        pltpu.VMEM((NCH, CH), jnp.int32),
        pltpu.VMEM((NCH, CH), jnp.int32),
        pltpu.VMEM((BPW, D), jnp.float32),
        pltpu.VMEM((BPW, D), jnp.float32),
        pltpu.VMEM((BPW, D), jnp.float32),
        pltpu.VMEM((BPW,), jnp.float32),
        pltpu.VMEM((BPW,), jnp.float32),
        pltpu.SemaphoreType.DMA,
    ],
    compiler_params=pltpu.CompilerParams(
        use_tc_tiling_on_sc=False, needs_layout_passes=False
    ),
)
def _bprmf_sc(user_hbm, itemi_hbm, itemj_hbm, ut_hbm, it_hbm,
              out_i, out_j,
              uix, iix, jix, urows, irows, jrows, oi, oj, sem):
    wid = lax.axis_index("s") * 2 + lax.axis_index("c")
    base = wid * BPW

    pltpu.sync_copy(user_hbm.at[wid], uix)
    pltpu.sync_copy(itemi_hbm.at[wid], iix)
    pltpu.sync_copy(itemj_hbm.at[wid], jix)

    copies = []
    for k in range(NCH):
        dst = pl.ds(k * CH, CH)
        copies.append(pltpu.async_copy(ut_hbm.at[uix.at[k]], urows.at[dst], sem))
        copies.append(pltpu.async_copy(it_hbm.at[iix.at[k]], irows.at[dst], sem))
        copies.append(pltpu.async_copy(it_hbm.at[jix.at[k]], jrows.at[dst], sem))
    for cp in copies:
        cp.wait()

    iota = jnp.arange(L, dtype=jnp.int32)

    def body(g, carry):
        rowids = g * L + iota
        acc_i = jnp.zeros((L,), jnp.float32)
        acc_j = jnp.zeros((L,), jnp.float32)
        for d in range(D):
            colids = jnp.full((L,), d, dtype=jnp.int32)
            u = plsc.load_gather(urows, [rowids, colids])
            acc_i = acc_i + u * plsc.load_gather(irows, [rowids, colids])
            acc_j = acc_j + u * plsc.load_gather(jrows, [rowids, colids])
        off = pl.multiple_of(g * L, L)
        oi[pl.ds(off, L)] = acc_i
        oj[pl.ds(off, L)] = acc_j
        return carry

    lax.fori_loop(0, BPW // L, body, 0)

    pltpu.sync_copy(oi, out_i.at[pl.ds(base, BPW)])
    pltpu.sync_copy(oj, out_j.at[pl.ds(base, BPW)])


def kernel(user, item_i, item_j, user_table, item_table):
    user_r = user.astype(jnp.int32).reshape(NW, NCH, CH)
    itemi_r = item_i.astype(jnp.int32).reshape(NW, NCH, CH)
    itemj_r = item_j.astype(jnp.int32).reshape(NW, NCH, CH)
    one = (user[0] * 0 + 1).astype(jnp.float32)
    ut = user_table * one
    it = item_table * one
    return _bprmf_sc(user_r, itemi_r, itemj_r, ut, it)
